# Initial kernel scaffold; baseline (speedup 1.0000x reference)
import jax
import jax.numpy as jnp
from jax.experimental import pallas as pl


def kernel(x, adj, W1, b1, gamma1, beta1, W2, b2, gamma2, beta2):
    raise NotImplementedError("write your pallas kernel here")


# speedup vs baseline: 1.0125x; 1.0125x over previous
import functools

import jax
import jax.numpy as jnp
from jax.experimental import pallas as pl
from jax.experimental.pallas import tpu as pltpu

_EPS = 1e-5


def _spmm_stats_body(adj_ref, s_ref, h_ref, sum_ref, sq_ref):
    i = pl.program_id(0)
    a = adj_ref[...].astype(jnp.bfloat16)
    h = jnp.dot(a, s_ref[...], preferred_element_type=jnp.float32)
    h_ref[...] = h

    @pl.when(i == 0)
    def _init():
        sum_ref[...] = jnp.zeros_like(sum_ref)
        sq_ref[...] = jnp.zeros_like(sq_ref)

    sum_ref[...] += jnp.sum(h, axis=0, keepdims=True)
    sq_ref[...] += jnp.sum(h * h, axis=0, keepdims=True)


def _spmm_stats(adj, s_bf16, block_rows):
    n = adj.shape[0]
    f = s_bf16.shape[1]
    return pl.pallas_call(
        _spmm_stats_body,
        grid=(n // block_rows,),
        in_specs=[
            pl.BlockSpec((block_rows, n), lambda i: (i, 0)),
            pl.BlockSpec((n, f), lambda i: (0, 0)),
        ],
        out_specs=[
            pl.BlockSpec((block_rows, f), lambda i: (i, 0)),
            pl.BlockSpec((1, f), lambda i: (0, 0)),
            pl.BlockSpec((1, f), lambda i: (0, 0)),
        ],
        out_shape=[
            jax.ShapeDtypeStruct((n, f), jnp.float32),
            jax.ShapeDtypeStruct((1, f), jnp.float32),
            jax.ShapeDtypeStruct((1, f), jnp.float32),
        ],
        compiler_params=pltpu.CompilerParams(
            dimension_semantics=("arbitrary",),
        ),
    )(adj, s_bf16)


def _affine_mm_body(h_ref, w_ref, a_ref, c_ref, o_ref):
    h = h_ref[...] * a_ref[...] + c_ref[...]
    o_ref[...] = jnp.dot(
        h, w_ref[...], preferred_element_type=jnp.float32
    ).astype(o_ref.dtype)


def _affine_mm_bf16(h, w, a, c, block_rows):
    n, f_in = h.shape
    f_out = w.shape[1]
    return pl.pallas_call(
        _affine_mm_body,
        grid=(n // block_rows,),
        in_specs=[
            pl.BlockSpec((block_rows, f_in), lambda i: (i, 0)),
            pl.BlockSpec((f_in, f_out), lambda i: (0, 0)),
            pl.BlockSpec((1, f_in), lambda i: (0, 0)),
            pl.BlockSpec((1, f_in), lambda i: (0, 0)),
        ],
        out_specs=pl.BlockSpec((block_rows, f_out), lambda i: (i, 0)),
        out_shape=jax.ShapeDtypeStruct((n, f_out), jnp.bfloat16),
    )(h, w, a, c)


def _bn_tanh_body(h_ref, a_ref, c_ref, o_ref):
    o_ref[...] = jnp.tanh(h_ref[...] * a_ref[...] + c_ref[...])


def _bn_tanh(h, a, c, block_rows):
    n, f = h.shape
    return pl.pallas_call(
        _bn_tanh_body,
        grid=(n // block_rows,),
        in_specs=[
            pl.BlockSpec((block_rows, f), lambda i: (i, 0)),
            pl.BlockSpec((1, f), lambda i: (0, 0)),
            pl.BlockSpec((1, f), lambda i: (0, 0)),
        ],
        out_specs=pl.BlockSpec((block_rows, f), lambda i: (i, 0)),
        out_shape=jax.ShapeDtypeStruct((n, f), jnp.float32),
    )(h, a, c)


def _bn_coeffs(s, q, n, gamma, beta):
    m = s / n
    v = q / n - m * m
    a = (gamma * jax.lax.rsqrt(v + _EPS)).reshape(1, -1)
    c = (beta - m.reshape(-1) * a.reshape(-1)).reshape(1, -1)
    return a, c


def kernel(x, adj, W1, b1, gamma1, beta1, W2, b2, gamma2, beta2):
    n, f_in = x.shape
    big_block = 200 if n % 200 == 0 else 8
    small_block = 2000 if n % 2000 == 0 else 8

    ones = jnp.ones((1, f_in), jnp.float32)
    zeros = jnp.zeros((1, f_in), jnp.float32)

    s1 = _affine_mm_bf16(x, W1, ones, zeros, small_block)
    h1, st_s1, st_q1 = _spmm_stats(adj, s1, big_block)
    a1, c1 = _bn_coeffs(st_s1, st_q1, n, gamma1, beta1)

    m2 = _affine_mm_bf16(h1, W2, a1, c1, small_block)
    h2, st_s2, st_q2 = _spmm_stats(adj, m2, big_block)
    a2, c2 = _bn_coeffs(st_s2, st_q2, n, gamma2, beta2)

    return _bn_tanh(h2, a2, c2, small_block)


# speedup vs baseline: 1.0170x; 1.0044x over previous
import functools

import jax
import jax.numpy as jnp
from jax.experimental import pallas as pl
from jax.experimental.pallas import tpu as pltpu

_EPS = 1e-5


def _spmm_stats_body(adj_ref, s_ref, h_ref, sum_ref, sq_ref):
    i = pl.program_id(0)
    a = adj_ref[...].astype(jnp.bfloat16)
    h = jnp.dot(a, s_ref[...], preferred_element_type=jnp.float32)
    h_ref[...] = h

    @pl.when(i == 0)
    def _init():
        sum_ref[...] = jnp.zeros_like(sum_ref)
        sq_ref[...] = jnp.zeros_like(sq_ref)

    sum_ref[...] += jnp.sum(h, axis=0, keepdims=True)
    sq_ref[...] += jnp.sum(h * h, axis=0, keepdims=True)


def _spmm_stats(adj, s_bf16, block_rows):
    n = adj.shape[0]
    f = s_bf16.shape[1]
    return pl.pallas_call(
        _spmm_stats_body,
        grid=(n // block_rows,),
        in_specs=[
            pl.BlockSpec((block_rows, n), lambda i: (i, 0)),
            pl.BlockSpec((n, f), lambda i: (0, 0)),
        ],
        out_specs=[
            pl.BlockSpec((block_rows, f), lambda i: (i, 0)),
            pl.BlockSpec((1, f), lambda i: (0, 0)),
            pl.BlockSpec((1, f), lambda i: (0, 0)),
        ],
        out_shape=[
            jax.ShapeDtypeStruct((n, f), jnp.float32),
            jax.ShapeDtypeStruct((1, f), jnp.float32),
            jax.ShapeDtypeStruct((1, f), jnp.float32),
        ],
        compiler_params=pltpu.CompilerParams(
            dimension_semantics=("arbitrary",),
            vmem_limit_bytes=100 * 1024 * 1024,
        ),
    )(adj, s_bf16)


def _affine_mm_body(h_ref, w_ref, a_ref, c_ref, o_ref):
    h = h_ref[...] * a_ref[...] + c_ref[...]
    o_ref[...] = jnp.dot(
        h, w_ref[...], preferred_element_type=jnp.float32
    ).astype(o_ref.dtype)


def _affine_mm_bf16(h, w, a, c, block_rows):
    n, f_in = h.shape
    f_out = w.shape[1]
    return pl.pallas_call(
        _affine_mm_body,
        grid=(n // block_rows,),
        in_specs=[
            pl.BlockSpec((block_rows, f_in), lambda i: (i, 0)),
            pl.BlockSpec((f_in, f_out), lambda i: (0, 0)),
            pl.BlockSpec((1, f_in), lambda i: (0, 0)),
            pl.BlockSpec((1, f_in), lambda i: (0, 0)),
        ],
        out_specs=pl.BlockSpec((block_rows, f_out), lambda i: (i, 0)),
        out_shape=jax.ShapeDtypeStruct((n, f_out), jnp.bfloat16),
    )(h, w, a, c)


def _bn_tanh_body(h_ref, a_ref, c_ref, o_ref):
    o_ref[...] = jnp.tanh(h_ref[...] * a_ref[...] + c_ref[...])


def _bn_tanh(h, a, c, block_rows):
    n, f = h.shape
    return pl.pallas_call(
        _bn_tanh_body,
        grid=(n // block_rows,),
        in_specs=[
            pl.BlockSpec((block_rows, f), lambda i: (i, 0)),
            pl.BlockSpec((1, f), lambda i: (0, 0)),
            pl.BlockSpec((1, f), lambda i: (0, 0)),
        ],
        out_specs=pl.BlockSpec((block_rows, f), lambda i: (i, 0)),
        out_shape=jax.ShapeDtypeStruct((n, f), jnp.float32),
    )(h, a, c)


def _bn_coeffs(s, q, n, gamma, beta):
    m = s / n
    v = q / n - m * m
    a = (gamma * jax.lax.rsqrt(v + _EPS)).reshape(1, -1)
    c = (beta - m.reshape(-1) * a.reshape(-1)).reshape(1, -1)
    return a, c


def kernel(x, adj, W1, b1, gamma1, beta1, W2, b2, gamma2, beta2):
    n, f_in = x.shape
    big_block = 400 if n % 400 == 0 else 8
    small_block = 2000 if n % 2000 == 0 else 8

    ones = jnp.ones((1, f_in), jnp.float32)
    zeros = jnp.zeros((1, f_in), jnp.float32)

    s1 = _affine_mm_bf16(x, W1, ones, zeros, small_block)
    h1, st_s1, st_q1 = _spmm_stats(adj, s1, big_block)
    a1, c1 = _bn_coeffs(st_s1, st_q1, n, gamma1, beta1)

    m2 = _affine_mm_bf16(h1, W2, a1, c1, small_block)
    h2, st_s2, st_q2 = _spmm_stats(adj, m2, big_block)
    a2, c2 = _bn_coeffs(st_s2, st_q2, n, gamma2, beta2)

    return _bn_tanh(h2, a2, c2, small_block)
